# baseline (speedup 1.0000x reference)
import functools

import jax
import jax.numpy as jnp
from jax.experimental import pallas as pl

_EMBED_DIM = 64
_NUM_EMB = 1024
_BLK = 2048
_CHUNK = 256


def _proj_body(emb_ref, w_out_ref, bout_ref, p_ref):
    p_ref[...] = (jnp.sum(emb_ref[...] * w_out_ref[...], axis=1, keepdims=True).T
                  + bout_ref[...])


def _vq_body(x_ref, embT2_ref, es_ref, w_in_ref, b_in_ref, p_ref,
             out_ref, loss_ref):
    i = pl.program_id(0)
    w_in = w_in_ref[...]
    b_in = b_in_ref[...]
    es = es_ref[...]
    p_row = p_ref[...]
    embT2 = embT2_ref[...]

    def chunk(k, acc):
        x_col = x_ref[pl.ds(k * _CHUNK, _CHUNK), :]
        h = x_col * w_in + b_in
        m2 = jnp.dot(h, embT2)
        fs = jnp.sum(h * h, axis=1, keepdims=True)
        dist = fs - m2 + es
        minval = jnp.min(dist, axis=1, keepdims=True)
        out_ref[pl.ds(k * _CHUNK, _CHUNK), :] = jnp.sum(
            jnp.where(dist == minval, p_row, 0.0), axis=1, keepdims=True)
        return acc + jnp.sum(minval, axis=0, keepdims=True)

    blk_err = jax.lax.fori_loop(0, _BLK // _CHUNK, chunk,
                                jnp.zeros((1, 1), jnp.float32))

    @pl.when(i == 0)
    def _():
        loss_ref[...] = jnp.zeros_like(loss_ref)

    loss_ref[...] += blk_err


@functools.partial(jax.jit, static_argnames=())
def kernel(x, W_in, b_in, W_out, b_out, emb):
    B, C, H, W = x.shape
    n = B * C * H * W
    nblk = n // _BLK

    proj = pl.pallas_call(
        _proj_body,
        out_shape=jax.ShapeDtypeStruct((1, _NUM_EMB), jnp.float32),
    )(emb, W_out.reshape(1, _EMBED_DIM), b_out.reshape(1, 1))

    embT2 = emb.T + emb.T
    es = jnp.sum(emb * emb, axis=1)[None, :]

    x2 = x.reshape(n, 1)
    out2, loss_sum = pl.pallas_call(
        _vq_body,
        grid=(nblk,),
        in_specs=[
            pl.BlockSpec((_BLK, 1), lambda i: (i, 0)),
            pl.BlockSpec((_EMBED_DIM, _NUM_EMB), lambda i: (0, 0)),
            pl.BlockSpec((1, _NUM_EMB), lambda i: (0, 0)),
            pl.BlockSpec((1, _EMBED_DIM), lambda i: (0, 0)),
            pl.BlockSpec((1, _EMBED_DIM), lambda i: (0, 0)),
            pl.BlockSpec((1, _NUM_EMB), lambda i: (0, 0)),
        ],
        out_specs=[
            pl.BlockSpec((_BLK, 1), lambda i: (i, 0)),
            pl.BlockSpec((1, 1), lambda i: (0, 0)),
        ],
        out_shape=(
            jax.ShapeDtypeStruct((n, 1), jnp.float32),
            jax.ShapeDtypeStruct((1, 1), jnp.float32),
        ),
    )(x2, embT2, es, W_in.reshape(1, _EMBED_DIM), b_in.reshape(1, _EMBED_DIM),
      proj)

    out = out2.reshape(B, C, H, W)
    emb_loss = (10.0 * (1.0 + 0.25) / (n * _EMBED_DIM)) * loss_sum[0, 0]
    return out, emb_loss


# speedup vs baseline: 1.3132x; 1.3132x over previous
import functools

import jax
import jax.numpy as jnp
from jax.experimental import pallas as pl

_EMBED_DIM = 64
_NUM_EMB = 1024
_BLK = 8192


def _vq_body(x_ref, fs_ref, embT2_ref, es_ref, w_in_ref, b_in_ref, p_ref,
             out_ref, loss_ref):
    i = pl.program_id(0)
    x_col = x_ref[...]
    h = x_col * w_in_ref[...] + b_in_ref[...]
    m2 = jnp.dot(h, embT2_ref[...])
    dist = fs_ref[...] - m2 + es_ref[...]
    minval = jnp.min(dist, axis=1, keepdims=True)
    out_ref[...] = jnp.sum(jnp.where(dist == minval, p_ref[...], 0.0),
                           axis=1, keepdims=True)
    blk_err = jnp.sum(minval, axis=0, keepdims=True)

    @pl.when(i == 0)
    def _():
        loss_ref[...] = jnp.zeros_like(loss_ref)

    loss_ref[...] += blk_err


@functools.partial(jax.jit, static_argnames=())
def kernel(x, W_in, b_in, W_out, b_out, emb):
    B, C, H, W = x.shape
    n = B * C * H * W
    nblk = n // _BLK

    proj = (jnp.einsum('nd,d->n', emb, W_out) + b_out[0])[None, :]

    embT2 = emb.T + emb.T
    es = jnp.sum(emb * emb, axis=1)[None, :]

    x2 = x.reshape(n, 1)
    h_flat = x2 * W_in.reshape(1, _EMBED_DIM) + b_in.reshape(1, _EMBED_DIM)
    fs = jnp.sum(h_flat * h_flat, axis=1, keepdims=True)
    out2, loss_sum = pl.pallas_call(
        _vq_body,
        grid=(nblk,),
        in_specs=[
            pl.BlockSpec((_BLK, 1), lambda i: (i, 0)),
            pl.BlockSpec((_BLK, 1), lambda i: (i, 0)),
            pl.BlockSpec((_EMBED_DIM, _NUM_EMB), lambda i: (0, 0)),
            pl.BlockSpec((1, _NUM_EMB), lambda i: (0, 0)),
            pl.BlockSpec((1, _EMBED_DIM), lambda i: (0, 0)),
            pl.BlockSpec((1, _EMBED_DIM), lambda i: (0, 0)),
            pl.BlockSpec((1, _NUM_EMB), lambda i: (0, 0)),
        ],
        out_specs=[
            pl.BlockSpec((_BLK, 1), lambda i: (i, 0)),
            pl.BlockSpec((1, 1), lambda i: (0, 0)),
        ],
        out_shape=(
            jax.ShapeDtypeStruct((n, 1), jnp.float32),
            jax.ShapeDtypeStruct((1, 1), jnp.float32),
        ),
    )(x2, fs, embT2, es, W_in.reshape(1, _EMBED_DIM), b_in.reshape(1, _EMBED_DIM),
      proj)

    out = out2.reshape(B, C, H, W)
    emb_loss = (10.0 * (1.0 + 0.25) / (n * _EMBED_DIM)) * loss_sum[0, 0]
    return out, emb_loss


# speedup vs baseline: 1.6492x; 1.2559x over previous
import functools

import jax
import jax.numpy as jnp
from jax.experimental import pallas as pl

_EMBED_DIM = 64
_NUM_EMB = 1024
_BLK = 8192


def _vq_body(x_ref, embT2_ref, es_ref, w_in_ref, b_in_ref, p_ref,
             out_ref, loss_ref):
    i = pl.program_id(0)
    x_col = x_ref[...]
    h = x_col * w_in_ref[...] + b_in_ref[...]
    m2 = jnp.dot(h, embT2_ref[...])
    fs = jnp.sum(h * h, axis=1, keepdims=True)
    dist = fs - m2 + es_ref[...]
    minval = jnp.min(dist, axis=1, keepdims=True)
    out_ref[...] = jnp.sum(jnp.where(dist == minval, p_ref[...], 0.0),
                           axis=1, keepdims=True)
    blk_err = jnp.sum(minval, axis=0, keepdims=True)

    @pl.when(i == 0)
    def _():
        loss_ref[...] = jnp.zeros_like(loss_ref)

    loss_ref[...] += blk_err


@functools.partial(jax.jit, static_argnames=())
def kernel(x, W_in, b_in, W_out, b_out, emb):
    B, C, H, W = x.shape
    n = B * C * H * W
    nblk = n // _BLK

    proj = (jnp.einsum('nd,d->n', emb, W_out) + b_out[0])[None, :]

    embT2 = emb.T + emb.T
    es = jnp.sum(emb * emb, axis=1)[None, :]

    x2 = x.reshape(n, 1)
    out2, loss_sum = pl.pallas_call(
        _vq_body,
        grid=(nblk,),
        in_specs=[
            pl.BlockSpec((_BLK, 1), lambda i: (i, 0)),
            pl.BlockSpec((_EMBED_DIM, _NUM_EMB), lambda i: (0, 0)),
            pl.BlockSpec((1, _NUM_EMB), lambda i: (0, 0)),
            pl.BlockSpec((1, _EMBED_DIM), lambda i: (0, 0)),
            pl.BlockSpec((1, _EMBED_DIM), lambda i: (0, 0)),
            pl.BlockSpec((1, _NUM_EMB), lambda i: (0, 0)),
        ],
        out_specs=[
            pl.BlockSpec((_BLK, 1), lambda i: (i, 0)),
            pl.BlockSpec((1, 1), lambda i: (0, 0)),
        ],
        out_shape=(
            jax.ShapeDtypeStruct((n, 1), jnp.float32),
            jax.ShapeDtypeStruct((1, 1), jnp.float32),
        ),
    )(x2, embT2, es, W_in.reshape(1, _EMBED_DIM), b_in.reshape(1, _EMBED_DIM),
      proj)

    out = out2.reshape(B, C, H, W)
    emb_loss = (10.0 * (1.0 + 0.25) / (n * _EMBED_DIM)) * loss_sum[0, 0]
    return out, emb_loss
